# baseline (speedup 1.0000x reference)
import functools

import jax
import jax.numpy as jnp
from jax.experimental import pallas as pl

N = 4096
BLOCK_ROWS = 256
CHUNK = 256


def _cumsum_block_kernel(x_ref, o_ref, *, chunk):
    x = x_ref[...]
    rows, n = x.shape
    nchunks = n // chunk
    col = jax.lax.broadcasted_iota(jnp.int32, (chunk, chunk), 1)
    row = jax.lax.broadcasted_iota(jnp.int32, (chunk, chunk), 0)
    tri = (row <= col).astype(jnp.bfloat16)
    hi = x.astype(jnp.bfloat16)
    lo = (x - hi.astype(jnp.float32)).astype(jnp.bfloat16)
    carry = jnp.zeros((rows, 1), jnp.float32)
    for c in range(nchunks):
        sl = pl.ds(c * chunk, chunk)
        cs = (
            jax.lax.dot(hi[:, c * chunk:(c + 1) * chunk], tri,
                        preferred_element_type=jnp.float32)
            + jax.lax.dot(lo[:, c * chunk:(c + 1) * chunk], tri,
                          preferred_element_type=jnp.float32)
            + carry
        )
        o_ref[:, sl] = cs
        carry = cs[:, chunk - 1:chunk]


def kernel(x):
    rows, n = x.shape
    grid = (rows // BLOCK_ROWS,)
    return pl.pallas_call(
        functools.partial(_cumsum_block_kernel, chunk=CHUNK),
        grid=grid,
        in_specs=[pl.BlockSpec((BLOCK_ROWS, n), lambda i: (i, 0))],
        out_specs=pl.BlockSpec((BLOCK_ROWS, n), lambda i: (i, 0)),
        out_shape=jax.ShapeDtypeStruct((rows, n), jnp.float32),
    )(x)


# speedup vs baseline: 1.0344x; 1.0344x over previous
import functools

import jax
import jax.numpy as jnp
from jax.experimental import pallas as pl
from jax.experimental.pallas import tpu as pltpu

N = 4096
BLOCK_ROWS = 512
CHUNK = 256


def _cumsum_block_kernel(x_ref, o_ref, *, chunk):
    x = x_ref[...]
    rows, n = x.shape
    nchunks = n // chunk
    col = jax.lax.broadcasted_iota(jnp.int32, (chunk, chunk), 1)
    row = jax.lax.broadcasted_iota(jnp.int32, (chunk, chunk), 0)
    tri = (row <= col).astype(jnp.bfloat16)
    hi = x.astype(jnp.bfloat16)
    lo = (x - hi.astype(jnp.float32)).astype(jnp.bfloat16)
    carry = jnp.zeros((rows, 1), jnp.float32)
    for c in range(nchunks):
        sl = pl.ds(c * chunk, chunk)
        cs = (
            jax.lax.dot(hi[:, c * chunk:(c + 1) * chunk], tri,
                        preferred_element_type=jnp.float32)
            + jax.lax.dot(lo[:, c * chunk:(c + 1) * chunk], tri,
                          preferred_element_type=jnp.float32)
            + carry
        )
        o_ref[:, sl] = cs
        carry = cs[:, chunk - 1:chunk]


def kernel(x):
    rows, n = x.shape
    grid = (rows // BLOCK_ROWS,)
    return pl.pallas_call(
        functools.partial(_cumsum_block_kernel, chunk=CHUNK),
        grid=grid,
        in_specs=[pl.BlockSpec((BLOCK_ROWS, n), lambda i: (i, 0))],
        out_specs=pl.BlockSpec((BLOCK_ROWS, n), lambda i: (i, 0)),
        out_shape=jax.ShapeDtypeStruct((rows, n), jnp.float32),
        compiler_params=pltpu.CompilerParams(
            dimension_semantics=("parallel",),
        ),
    )(x)


# speedup vs baseline: 1.0875x; 1.0514x over previous
import functools

import jax
import jax.numpy as jnp
from jax.experimental import pallas as pl
from jax.experimental.pallas import tpu as pltpu

N = 4096
BLOCK_ROWS = 512
CHUNK = 256


def _cumsum_block_kernel(x_ref, o_ref, *, chunk):
    x = x_ref[...]
    rows, n = x.shape
    nchunks = n // chunk
    col = jax.lax.broadcasted_iota(jnp.int32, (chunk, chunk), 1)
    row = jax.lax.broadcasted_iota(jnp.int32, (chunk, chunk), 0)
    tri = (row <= col).astype(jnp.bfloat16)
    hi = x.astype(jnp.bfloat16)
    carry = jnp.zeros((rows, 1), jnp.float32)
    for c in range(nchunks):
        sl = pl.ds(c * chunk, chunk)
        cs = (
            jax.lax.dot(hi[:, c * chunk:(c + 1) * chunk], tri,
                        preferred_element_type=jnp.float32)
            + carry
        )
        o_ref[:, sl] = cs
        carry = cs[:, chunk - 1:chunk]


def kernel(x):
    rows, n = x.shape
    grid = (rows // BLOCK_ROWS,)
    return pl.pallas_call(
        functools.partial(_cumsum_block_kernel, chunk=CHUNK),
        grid=grid,
        in_specs=[pl.BlockSpec((BLOCK_ROWS, n), lambda i: (i, 0))],
        out_specs=pl.BlockSpec((BLOCK_ROWS, n), lambda i: (i, 0)),
        out_shape=jax.ShapeDtypeStruct((rows, n), jnp.float32),
        compiler_params=pltpu.CompilerParams(
            dimension_semantics=("parallel",),
        ),
    )(x)
